# baseline (speedup 1.0000x reference)
import jax
import jax.numpy as jnp
from jax.experimental import pallas as pl
from jax.experimental.pallas import tpu as pltpu

_TOPK = 8
_TD = 256
_NT = 11


def _router_kernel(x_ref, w_ref, ids_ref, p_ref):
    pooled = jnp.mean(x_ref[...], axis=0, keepdims=True)
    logits = jnp.dot(pooled, w_ref[...],
                     preferred_element_type=jnp.float32)
    e = logits.shape[1]
    iota = jax.lax.broadcasted_iota(jnp.int32, (1, e), 1)
    iota_k = jax.lax.broadcasted_iota(jnp.int32, (1, _TOPK), 1)
    neg_inf = jnp.float32(-jnp.inf)

    vals = logits
    ids_acc = jnp.zeros((1, _TOPK), jnp.int32)
    sel = jnp.zeros((1, e), jnp.bool_)
    for i in range(_TOPK):
        m = jnp.max(vals)
        idx = jnp.min(jnp.where(vals == m, iota, e))
        ids_acc = jnp.where(iota_k == i, idx, ids_acc)
        sel = sel | (iota == idx)
        vals = jnp.where(iota == idx, neg_inf, vals)

    z = jnp.where(sel, logits, neg_inf)
    zmax = jnp.max(z)
    ez = jnp.where(sel, jnp.exp(z - zmax), 0.0)
    p = ez / jnp.sum(ez)
    ids_ref[...] = ids_acc
    p_ref[...] = p


def _moe_kernel(ids_ref, p_ref, x_ref, g_ref, u_ref, d_ref, o_ref,
                inter_ref, dwb_ref):
    k = pl.program_id(0)
    ph = pl.program_id(1)

    @pl.when(ph < _NT)
    def _():
        prob = p_ref[ids_ref[k]]
        xb = x_ref[...]
        gwb = g_ref[0].astype(jnp.bfloat16)
        uwb = u_ref[0].astype(jnp.bfloat16)
        gate = jax.lax.dot_general(xb, gwb, (((1,), (1,)), ((), ())),
                                   preferred_element_type=jnp.float32)
        up = jax.lax.dot_general(xb, uwb, (((1,), (1,)), ((), ())),
                                 preferred_element_type=jnp.float32)
        inter = (jax.nn.silu(gate) * up).astype(jnp.bfloat16)
        dwt = (d_ref[0] * prob).astype(jnp.bfloat16)
        for t in range(_NT):
            @pl.when(ph == t)
            def _():
                inter_ref[:, t * _TD:(t + 1) * _TD] = inter
                dwb_ref[:, t * _TD:(t + 1) * _TD] = dwt

    @pl.when(ph == _NT)
    def _():
        y = jax.lax.dot_general(inter_ref[...], dwb_ref[...],
                                (((1,), (1,)), ((), ())),
                                preferred_element_type=jnp.float32)

        @pl.when(k == 0)
        def _():
            o_ref[...] = y

        @pl.when(k != 0)
        def _():
            o_ref[...] += y


def kernel(hidden_states, W_router, gate_w, up_w, down_w):
    b, s, h = hidden_states.shape
    e, dff, _ = gate_w.shape
    m = b * s
    x2d = hidden_states.reshape(m, h)

    ids2d, p2d = pl.pallas_call(
        _router_kernel,
        out_shape=(
            jax.ShapeDtypeStruct((1, _TOPK), jnp.int32),
            jax.ShapeDtypeStruct((1, e), jnp.float32),
        ),
    )(x2d, W_router)
    ids = ids2d[0]
    probs = p2d[0]

    xb = x2d.astype(jnp.bfloat16)
    n_dff = dff // _TD
    grid = (_TOPK, n_dff + 1)
    last = n_dff - 1
    out2d = pl.pallas_call(
        _moe_kernel,
        grid_spec=pltpu.PrefetchScalarGridSpec(
            num_scalar_prefetch=2,
            grid=grid,
            in_specs=[
                pl.BlockSpec((m, h), lambda k, ph, ids, p: (0, 0)),
                pl.BlockSpec((1, _TD, h),
                             lambda k, ph, ids, p: (ids[k], jnp.minimum(ph, last), 0)),
                pl.BlockSpec((1, _TD, h),
                             lambda k, ph, ids, p: (ids[k], jnp.minimum(ph, last), 0)),
                pl.BlockSpec((1, h, _TD),
                             lambda k, ph, ids, p: (ids[k], 0, jnp.minimum(ph, last))),
            ],
            out_specs=pl.BlockSpec((m, h), lambda k, ph, ids, p: (0, 0)),
            scratch_shapes=[
                pltpu.VMEM((m, dff), jnp.bfloat16),
                pltpu.VMEM((h, dff), jnp.bfloat16),
            ],
        ),
        out_shape=jax.ShapeDtypeStruct((m, h), jnp.float32),
        compiler_params=pltpu.CompilerParams(
            dimension_semantics=("arbitrary", "arbitrary"),
        ),
    )(ids, probs, xb, gate_w, up_w, down_w)

    return out2d.reshape(b, s, h)


# speedup vs baseline: 1.0221x; 1.0221x over previous
import jax
import jax.numpy as jnp
from jax.experimental import pallas as pl
from jax.experimental.pallas import tpu as pltpu

_TOPK = 8
_TD = 256
_NT = 11


def _router_kernel(x_ref, w_ref, ids_ref, p_ref):
    pooled = jnp.mean(x_ref[...], axis=0, keepdims=True)
    logits = jnp.dot(pooled, w_ref[...],
                     preferred_element_type=jnp.float32)
    e = logits.shape[1]
    iota = jax.lax.broadcasted_iota(jnp.int32, (1, e), 1)
    iota_k = jax.lax.broadcasted_iota(jnp.int32, (1, _TOPK), 1)
    neg_inf = jnp.float32(-jnp.inf)

    vals = logits
    ids_acc = jnp.zeros((1, _TOPK), jnp.int32)
    sel = jnp.zeros((1, e), jnp.bool_)
    for i in range(_TOPK):
        m = jnp.max(vals)
        idx = jnp.min(jnp.where(vals == m, iota, e))
        ids_acc = jnp.where(iota_k == i, idx, ids_acc)
        sel = sel | (iota == idx)
        vals = jnp.where(iota == idx, neg_inf, vals)

    z = jnp.where(sel, logits, neg_inf)
    zmax = jnp.max(z)
    ez = jnp.where(sel, jnp.exp(z - zmax), 0.0)
    p = ez / jnp.sum(ez)
    ids_ref[...] = ids_acc
    p_ref[...] = p


def _moe_kernel(ids_ref, p_ref, x_ref, g_ref, u_ref, d_ref, o_ref,
                inter_ref, dwb_ref, guwb_ref):
    k = pl.program_id(0)
    ph = pl.program_id(1)

    @pl.when(ph < _NT)
    def _():
        guwb_ref[:_TD] = g_ref[0].astype(jnp.bfloat16)
        guwb_ref[_TD:] = u_ref[0].astype(jnp.bfloat16)
        xb = x_ref[...]
        gu = jax.lax.dot_general(xb, guwb_ref[...], (((1,), (1,)), ((), ())),
                                 preferred_element_type=jnp.float32)
        inter = jax.nn.silu(gu[:, :_TD]) * gu[:, _TD:]
        inter_ref[ph] = inter.astype(jnp.bfloat16)
        dwb_ref[ph] = d_ref[0].astype(jnp.bfloat16)

    @pl.when(ph == _NT)
    def _():
        prob = p_ref[ids_ref[k]]
        y = jax.lax.dot_general(inter_ref[0], dwb_ref[0],
                                (((1,), (1,)), ((), ())),
                                preferred_element_type=jnp.float32)
        for t in range(1, _NT):
            y = y + jax.lax.dot_general(inter_ref[t], dwb_ref[t],
                                        (((1,), (1,)), ((), ())),
                                        preferred_element_type=jnp.float32)

        @pl.when(k == 0)
        def _():
            o_ref[...] = y * prob

        @pl.when(k != 0)
        def _():
            o_ref[...] += y * prob


def kernel(hidden_states, W_router, gate_w, up_w, down_w):
    b, s, h = hidden_states.shape
    e, dff, _ = gate_w.shape
    m = b * s
    x2d = hidden_states.reshape(m, h)

    ids2d, p2d = pl.pallas_call(
        _router_kernel,
        out_shape=(
            jax.ShapeDtypeStruct((1, _TOPK), jnp.int32),
            jax.ShapeDtypeStruct((1, e), jnp.float32),
        ),
    )(x2d, W_router)
    ids = ids2d[0]
    probs = p2d[0]

    xb = x2d.astype(jnp.bfloat16)
    n_dff = dff // _TD
    grid = (_TOPK, n_dff + 1)
    last = n_dff - 1
    out2d = pl.pallas_call(
        _moe_kernel,
        grid_spec=pltpu.PrefetchScalarGridSpec(
            num_scalar_prefetch=2,
            grid=grid,
            in_specs=[
                pl.BlockSpec((m, h), lambda k, ph, ids, p: (0, 0)),
                pl.BlockSpec((1, _TD, h),
                             lambda k, ph, ids, p: (ids[k], jnp.minimum(ph, last), 0)),
                pl.BlockSpec((1, _TD, h),
                             lambda k, ph, ids, p: (ids[k], jnp.minimum(ph, last), 0)),
                pl.BlockSpec((1, h, _TD),
                             lambda k, ph, ids, p: (ids[k], 0, jnp.minimum(ph, last))),
            ],
            out_specs=pl.BlockSpec((m, h), lambda k, ph, ids, p: (0, 0)),
            scratch_shapes=[
                pltpu.VMEM((n_dff, m, _TD), jnp.bfloat16),
                pltpu.VMEM((n_dff, h, _TD), jnp.bfloat16),
                pltpu.VMEM((2 * _TD, h), jnp.bfloat16),
            ],
        ),
        out_shape=jax.ShapeDtypeStruct((m, h), jnp.float32),
        compiler_params=pltpu.CompilerParams(
            dimension_semantics=("arbitrary", "arbitrary"),
        ),
    )(ids, probs, xb, gate_w, up_w, down_w)

    return out2d.reshape(b, s, h)
